# baseline (speedup 1.0000x reference)
import functools
import math

import jax
import jax.numpy as jnp
from jax import lax
from jax.experimental import pallas as pl
from jax.experimental.pallas import tpu as pltpu
from jax.experimental.pallas import tpu_sc as plsc

D = 32
SCALE = math.sqrt(float(D))


def _gather_kernel(V, J, I, NC, NW):
    C = 512
    BPW = C // 128
    TPA = (I // 128) * 8 * 128
    assert NW * C == I
    mesh = plsc.VectorSubcoreMesh(core_axis_name="c", subcore_axis_name="s")

    @functools.partial(
        pl.kernel,
        mesh=mesh,
        out_type=jax.ShapeDtypeStruct((J * D * I,), jnp.float32),
        scratch_types=[
            pltpu.VMEM((C,), jnp.int32),
            pltpu.VMEM((C, D), jnp.float32),
            pltpu.VMEM((D * C,), jnp.float32),
            pltpu.SemaphoreType.DMA,
        ],
        compiler_params=pltpu.CompilerParams(
            use_tc_tiling_on_sc=False, needs_layout_passes=False),
    )
    def k(idx_hbm, tab_hbm, out_hbm, idx_v, rows_v, stage_v, sem):
        wid = lax.axis_index("s") * NC + lax.axis_index("c")
        iota = lax.iota(jnp.int32, 16)
        pos_lo = (iota // 8) * (BPW * 1024) + (iota % 8) * 128
        pos_hi = ((iota + 16) // 8) * (BPW * 1024) + (iota % 8) * 128

        def jloop(j, carry):
            base = j * I + wid * C
            pltpu.sync_copy(idx_hbm.at[pl.ds(base, C)], idx_v)
            pltpu.async_copy(tab_hbm.at[idx_v], rows_v, sem).wait()

            def block(b, cb):
                posb_lo = pos_lo + b * 1024
                posb_hi = pos_hi + b * 1024
                rbase = b * 128

                def row(l, c):
                    r = rbase + l
                    lo = rows_v[r, pl.ds(0, 16)]
                    hi = rows_v[r, pl.ds(16, 16)]
                    plsc.store_scatter(stage_v, [posb_lo + l], lo * SCALE)
                    plsc.store_scatter(stage_v, [posb_hi + l], hi * SCALE)
                    return c

                lax.fori_loop(0, 128, row, 0, unroll=8)
                return cb

            lax.fori_loop(0, BPW, block, 0)

            for a in range(D // 8):
                pltpu.sync_copy(
                    stage_v.at[pl.ds(a * BPW * 1024, BPW * 1024)],
                    out_hbm.at[pl.ds(j * D * I + a * TPA + wid * BPW * 1024,
                                     BPW * 1024)])
            return carry

        lax.fori_loop(0, J, jloop, 0)

    return k


def kernel(x, table):
    B0, B1 = x.shape
    V, d = table.shape
    I, J = B0, B1
    idx = x.T.reshape(J * I).astype(jnp.int32)
    info = plsc.get_sparse_core_info()
    NC = info.num_cores
    NW = NC * info.num_subcores
    flat = _gather_kernel(V, J, I, NC, NW)(idx, table)
    out5 = flat.reshape(J, d // 8, I // 128, 8, 128)
    return out5.transpose(2, 4, 0, 1, 3).reshape(I, J, d)


# speedup vs baseline: 1.1314x; 1.1314x over previous
import functools
import math

import jax
import jax.numpy as jnp
from jax import lax
from jax.experimental import pallas as pl
from jax.experimental.pallas import tpu as pltpu
from jax.experimental.pallas import tpu_sc as plsc

D = 32
SCALE = math.sqrt(float(D))


def _gather_kernel(V, J, I, NC, NW):
    C = 512
    BPW = C // 128
    TPA = (I // 128) * 8 * 128
    assert NW * C == I
    mesh = plsc.VectorSubcoreMesh(core_axis_name="c", subcore_axis_name="s")

    @functools.partial(
        pl.kernel,
        mesh=mesh,
        out_type=jax.ShapeDtypeStruct((J * D * I,), jnp.float32),
        scratch_types=[
            pltpu.VMEM((J, C), jnp.int32),
            pltpu.VMEM((2, C, D), jnp.float32),
            pltpu.VMEM((2, D * C), jnp.float32),
            pltpu.SemaphoreType.DMA,
            pltpu.SemaphoreType.DMA((2,)),
            pltpu.SemaphoreType.DMA((2,)),
        ],
        compiler_params=pltpu.CompilerParams(
            use_tc_tiling_on_sc=False, needs_layout_passes=False),
    )
    def k(idx_hbm, tab_hbm, out_hbm, idx_v, rows_v, stage_v, isem, gsem,
          osem):
        wid = lax.axis_index("s") * NC + lax.axis_index("c")
        iota = lax.iota(jnp.int32, 16)
        pos_lo = (iota // 8) * (BPW * 1024) + (iota % 8) * 128
        pos_hi = ((iota + 16) // 8) * (BPW * 1024) + (iota % 8) * 128

        def start_gather(j, buf):
            pltpu.async_copy(tab_hbm.at[idx_v.at[j]], rows_v.at[buf],
                             gsem.at[buf])

        def wait_gather(j, buf):
            pltpu.make_async_copy(tab_hbm.at[idx_v.at[j]], rows_v.at[buf],
                                  gsem.at[buf]).wait()

        def start_outs(j, buf):
            for a in range(D // 8):
                pltpu.async_copy(
                    stage_v.at[buf].at[pl.ds(a * BPW * 1024, BPW * 1024)],
                    out_hbm.at[pl.ds(j * D * I + a * TPA + wid * BPW * 1024,
                                     BPW * 1024)],
                    osem.at[buf])

        def wait_outs(j, buf):
            for a in range(D // 8):
                pltpu.make_async_copy(
                    stage_v.at[buf].at[pl.ds(a * BPW * 1024, BPW * 1024)],
                    out_hbm.at[pl.ds(j * D * I + a * TPA + wid * BPW * 1024,
                                     BPW * 1024)],
                    osem.at[buf]).wait()

        pltpu.async_copy(idx_hbm.at[:, wid, :], idx_v, isem).wait()
        start_gather(0, 0)

        def gloop(g, carry):
            for b in range(2):
                j = g * 2 + b
                cur = b
                nxt = 1 - b

                @pl.when(j + 1 < J)
                def _():
                    start_gather(j + 1, nxt)

                wait_gather(j, cur)

                @pl.when(g > 0)
                def _():
                    wait_outs(j - 2, cur)

                def block(bb, cb):
                    posb_lo = pos_lo + bb * 1024
                    posb_hi = pos_hi + bb * 1024
                    rbase = bb * 128

                    def row(l, c):
                        r = rbase + l
                        lo = rows_v[cur, r, pl.ds(0, 16)]
                        hi = rows_v[cur, r, pl.ds(16, 16)]
                        plsc.store_scatter(stage_v.at[cur],
                                           [posb_lo + l], lo * SCALE)
                        plsc.store_scatter(stage_v.at[cur],
                                           [posb_hi + l], hi * SCALE)
                        return c

                    lax.fori_loop(0, 128, row, 0, unroll=8)
                    return cb

                lax.fori_loop(0, BPW, block, 0)
                start_outs(j, cur)
            return carry

        lax.fori_loop(0, J // 2, gloop, 0)
        wait_outs(J - 2, 0)
        wait_outs(J - 1, 1)

    return k


def kernel(x, table):
    B0, B1 = x.shape
    V, d = table.shape
    I, J = B0, B1
    info = plsc.get_sparse_core_info()
    NC = info.num_cores
    NW = NC * info.num_subcores
    C = I // NW
    idx = x.T.reshape(J, NW, C).astype(jnp.int32)
    flat = _gather_kernel(V, J, I, NC, NW)(idx, table)
    out5 = flat.reshape(J, d // 8, I // 128, 8, 128)
    return out5.transpose(2, 4, 0, 1, 3).reshape(I, J, d)
